# baseline (speedup 1.0000x reference)
import jax
import jax.numpy as jnp
from jax.experimental import pallas as pl
from jax.experimental.pallas import tpu as pltpu

_NBUF = 2
_IB = 400


def _plan(n):
    if n % _IB == 0 and n >= 4 * _IB:
        full = n // _IB - 1
        blocks = [(i * _IB, _IB) for i in range(full)]
        st = full * _IB
        for sz in (200, 104, 96):
            blocks.append((st, sz))
            st += sz
        assert st == n
        return blocks
    blocks = []
    st = 0
    while st < n:
        sz = min(_IB, n - st)
        blocks.append((st, sz))
        st += sz
    return blocks


def _gcn_body(blocks, feat_hbm, adj_hbm, w_ref, b_ref, out_hbm,
              fbuf, buf, obuf, s_ref, fsem, asem, osem):
    n_blocks = len(blocks)

    def adj_cp(i):
        st, sz = blocks[i]
        j = i % _NBUF
        return pltpu.make_async_copy(
            adj_hbm.at[pl.ds(st, sz), :], buf.at[j, pl.ds(0, sz), :],
            asem.at[j])

    def out_cp(i):
        st, sz = blocks[i]
        oj = i % 2
        return pltpu.make_async_copy(
            obuf.at[oj, pl.ds(0, sz), :], out_hbm.at[pl.ds(st, sz), :],
            osem.at[oj])

    for i in range(min(_NBUF, n_blocks)):
        adj_cp(i).start()
    fcp = pltpu.make_async_copy(feat_hbm, fbuf, fsem)
    fcp.start()
    fcp.wait()
    s_ref[...] = jnp.dot(
        fbuf[...].astype(jnp.bfloat16),
        w_ref[...].astype(jnp.bfloat16),
        preferred_element_type=jnp.float32,
    ).astype(jnp.bfloat16)

    for i, (st, sz) in enumerate(blocks):
        oj = i % 2
        adj_cp(i).wait()
        acc = jnp.dot(
            buf[i % _NBUF, :sz, :].astype(jnp.bfloat16),
            s_ref[...],
            preferred_element_type=jnp.float32,
        )
        if i >= 2:
            out_cp(i - 2).wait()
        obuf[oj, :sz, :] = acc + b_ref[...]
        out_cp(i).start()
        if i + _NBUF < n_blocks:
            adj_cp(i + _NBUF).start()

    for i in range(max(0, n_blocks - 2), n_blocks):
        out_cp(i).wait()


@jax.jit
def kernel(feat, adj, W, b):
    N, din = feat.shape
    dout = W.shape[1]
    b2 = b.reshape(1, dout)
    blocks = _plan(N)
    maxb = max(sz for _, sz in blocks)

    def body(feat_hbm, adj_hbm, w_ref, b_ref, out_hbm,
             fbuf, buf, obuf, s_ref, fsem, asem, osem):
        _gcn_body(blocks, feat_hbm, adj_hbm, w_ref, b_ref, out_hbm,
                  fbuf, buf, obuf, s_ref, fsem, asem, osem)

    out = pl.pallas_call(
        body,
        in_specs=[
            pl.BlockSpec(memory_space=pl.ANY),
            pl.BlockSpec(memory_space=pl.ANY),
            pl.BlockSpec(memory_space=pltpu.MemorySpace.VMEM),
            pl.BlockSpec(memory_space=pltpu.MemorySpace.VMEM),
        ],
        out_specs=pl.BlockSpec(memory_space=pl.ANY),
        out_shape=jax.ShapeDtypeStruct((N, dout), jnp.float32),
        scratch_shapes=[
            pltpu.VMEM((N, din), jnp.float32),
            pltpu.VMEM((_NBUF, maxb, N), jnp.float32),
            pltpu.VMEM((2, maxb, dout), jnp.float32),
            pltpu.VMEM((N, dout), jnp.bfloat16),
            pltpu.SemaphoreType.DMA,
            pltpu.SemaphoreType.DMA((_NBUF,)),
            pltpu.SemaphoreType.DMA((2,)),
        ],
        compiler_params=pltpu.CompilerParams(
            vmem_limit_bytes=64 * 1024 * 1024,
        ),
    )(feat, adj, W, b2)
    return out


# speedup vs baseline: 1.0012x; 1.0012x over previous
import jax
import jax.numpy as jnp
from jax.experimental import pallas as pl
from jax.experimental.pallas import tpu as pltpu

_NBUF = 2
_NSPLIT = 4
_IB = 400


def _plan(n):
    if n % _IB == 0 and n >= 4 * _IB:
        full = n // _IB - 1
        blocks = [(i * _IB, _IB) for i in range(full)]
        st = full * _IB
        for sz in (200, 104, 96):
            blocks.append((st, sz))
            st += sz
        assert st == n
        return blocks
    blocks = []
    st = 0
    while st < n:
        sz = min(_IB, n - st)
        blocks.append((st, sz))
        st += sz
    return blocks


def _gcn_body(blocks, feat_hbm, adj_hbm, w_ref, b_ref, out_hbm,
              fbuf, buf, obuf, s_ref, fsem, asem, osem):
    n_blocks = len(blocks)

    def adj_cps(i):
        st, sz = blocks[i]
        j = i % _NBUF
        chunk = -(-(sz // _NSPLIT) // 8) * 8
        cps = []
        off = 0
        k = 0
        while off < sz:
            csz = min(chunk, sz - off)
            cps.append(pltpu.make_async_copy(
                adj_hbm.at[pl.ds(st + off, csz), :],
                buf.at[j, pl.ds(off, csz), :],
                asem.at[j, k]))
            off += csz
            k += 1
        return cps

    def out_cp(i):
        st, sz = blocks[i]
        oj = i % 2
        return pltpu.make_async_copy(
            obuf.at[oj, pl.ds(0, sz), :], out_hbm.at[pl.ds(st, sz), :],
            osem.at[oj])

    for i in range(min(_NBUF, n_blocks)):
        for c in adj_cps(i):
            c.start()
    fcp = pltpu.make_async_copy(feat_hbm, fbuf, fsem)
    fcp.start()
    fcp.wait()
    s_ref[...] = jnp.dot(
        fbuf[...].astype(jnp.bfloat16),
        w_ref[...].astype(jnp.bfloat16),
        preferred_element_type=jnp.float32,
    ).astype(jnp.bfloat16)

    for i, (st, sz) in enumerate(blocks):
        oj = i % 2
        for c in adj_cps(i):
            c.wait()
        acc = jnp.dot(
            buf[i % _NBUF, :sz, :].astype(jnp.bfloat16),
            s_ref[...],
            preferred_element_type=jnp.float32,
        )
        if i >= 2:
            out_cp(i - 2).wait()
        obuf[oj, :sz, :] = acc + b_ref[...]
        out_cp(i).start()
        if i + _NBUF < n_blocks:
            for c in adj_cps(i + _NBUF):
                c.start()

    for i in range(max(0, n_blocks - 2), n_blocks):
        out_cp(i).wait()


@jax.jit
def kernel(feat, adj, W, b):
    N, din = feat.shape
    dout = W.shape[1]
    b2 = b.reshape(1, dout)
    blocks = _plan(N)
    maxb = max(sz for _, sz in blocks)

    def body(feat_hbm, adj_hbm, w_ref, b_ref, out_hbm,
             fbuf, buf, obuf, s_ref, fsem, asem, osem):
        _gcn_body(blocks, feat_hbm, adj_hbm, w_ref, b_ref, out_hbm,
                  fbuf, buf, obuf, s_ref, fsem, asem, osem)

    out = pl.pallas_call(
        body,
        in_specs=[
            pl.BlockSpec(memory_space=pl.ANY),
            pl.BlockSpec(memory_space=pl.ANY),
            pl.BlockSpec(memory_space=pltpu.MemorySpace.VMEM),
            pl.BlockSpec(memory_space=pltpu.MemorySpace.VMEM),
        ],
        out_specs=pl.BlockSpec(memory_space=pl.ANY),
        out_shape=jax.ShapeDtypeStruct((N, dout), jnp.float32),
        scratch_shapes=[
            pltpu.VMEM((N, din), jnp.float32),
            pltpu.VMEM((_NBUF, maxb, N), jnp.float32),
            pltpu.VMEM((2, maxb, dout), jnp.float32),
            pltpu.VMEM((N, dout), jnp.bfloat16),
            pltpu.SemaphoreType.DMA,
            pltpu.SemaphoreType.DMA((_NBUF, _NSPLIT)),
            pltpu.SemaphoreType.DMA((2,)),
        ],
        compiler_params=pltpu.CompilerParams(
            vmem_limit_bytes=64 * 1024 * 1024,
        ),
    )(feat, adj, W, b2)
    return out


# speedup vs baseline: 1.2651x; 1.2635x over previous
import jax
import jax.numpy as jnp
from jax.experimental import pallas as pl
from jax.experimental.pallas import tpu as pltpu


def _gcn_body(feat_ref, adj_ref, w_ref, b_ref, out_ref, s_ref):
    i = pl.program_id(0)

    @pl.when(i == 0)
    def _():
        s_ref[...] = jnp.dot(
            feat_ref[...].astype(jnp.bfloat16),
            w_ref[...].astype(jnp.bfloat16),
            preferred_element_type=jnp.float32,
        ).astype(jnp.bfloat16)

    acc = jnp.dot(
        adj_ref[...].astype(jnp.bfloat16),
        s_ref[...],
        preferred_element_type=jnp.float32,
    )
    out_ref[...] = acc + b_ref[...]


def _pick_block(n: int) -> int:
    if n >= 416:
        return 416
    for ib in (400, 200, 80, 40, 16, 8):
        if n % ib == 0:
            return ib
    return n


@jax.jit
def kernel(feat, adj, W, b):
    N, din = feat.shape
    dout = W.shape[1]
    ib = _pick_block(N)
    b2 = b.reshape(1, dout)

    out = pl.pallas_call(
        _gcn_body,
        grid=(pl.cdiv(N, ib),),
        in_specs=[
            pl.BlockSpec((N, din), lambda i: (0, 0)),
            pl.BlockSpec((ib, N), lambda i: (i, 0)),
            pl.BlockSpec((din, dout), lambda i: (0, 0)),
            pl.BlockSpec((1, dout), lambda i: (0, 0)),
        ],
        out_specs=pl.BlockSpec((ib, dout), lambda i: (i, 0)),
        out_shape=jax.ShapeDtypeStruct((N, dout), jnp.float32),
        scratch_shapes=[pltpu.VMEM((N, dout), jnp.bfloat16)],
        compiler_params=pltpu.CompilerParams(
            vmem_limit_bytes=64 * 1024 * 1024,
        ),
    )(feat, adj, W, b2)
    return out
